# baseline (speedup 1.0000x reference)
import functools

import jax
import jax.numpy as jnp
import numpy as np
from jax import lax
from jax.experimental import pallas as pl
from jax.experimental.pallas import tpu as pltpu
from jax.experimental.pallas import tpu_sc as plsc

B = 16384
EMBED = 8
NUM_FEATS = 7
DESC_IN = 1024
DESC_OUT = 64
H1 = 128
H2 = 128
OUT = 64


_NC = 2
_NS = 16
_NW = _NC * _NS
_BPW = B // _NW
_CAT = NUM_FEATS * EMBED
_NIDX = _BPW * NUM_FEATS
_L = 16
_G = _BPW // 8
_ROWPAD = 128
_NOUT = _BPW * _ROWPAD

_SIZES = (6, 6, 4, 4, 4, 4, 6)
_OFF8 = np.zeros(NUM_FEATS, np.int32)
_acc = 0
for _fi, _s in enumerate(_SIZES):
  _OFF8[_fi] = _acc * EMBED
  _acc += _s
_TAB_LEN = _acc * EMBED

_p = np.arange(8 * _CAT, dtype=np.int32)
_b = _p // _CAT
_r = _p % _CAT
_i = _r // EMBED
_d = _r % EMBED
_PIDX_H = (_b * NUM_FEATS + _i).astype(np.int32)
_DST_H = (_b * _ROWPAD + _r).astype(np.int32)
_FO_H = (_OFF8[_i[:112]] + _d[:112]).astype(np.int32)


def _sc_gather_body(pcat_hbm, tab_hbm, ph_hbm, dh_hbm, fh_hbm, out_hbm,
                    idx_v, tab_v, ph_v, dh_v, fh_v, rows_v):
  wid = lax.axis_index("s") * _NC + lax.axis_index("c")
  pltpu.sync_copy(pcat_hbm.at[wid], idx_v)
  pltpu.sync_copy(tab_hbm, tab_v)
  pltpu.sync_copy(ph_hbm, ph_v)
  pltpu.sync_copy(dh_hbm, dh_v)
  pltpu.sync_copy(fh_hbm, fh_v)

  def step(g, carry):
    p0 = g * (8 * NUM_FEATS)
    d0 = g * (8 * _ROWPAD)
    for t in range(4):
      for j in range(NUM_FEATS):
        c = t * NUM_FEATS + j
        pidx = ph_v[pl.ds(c * _L, _L)] + p0
        tv = plsc.load_gather(idx_v, [pidx])
        fo = fh_v[pl.ds(j * _L, _L)] + tv * EMBED
        val = plsc.load_gather(tab_v, [fo])
        dst = dh_v[pl.ds(c * _L, _L)] + d0
        plsc.store_scatter(rows_v, [dst], val)
    return carry

  lax.fori_loop(0, _G, step, 0)
  pltpu.sync_copy(rows_v, out_hbm.at[pl.ds(wid * _NOUT, _NOUT)])


@functools.cache
def _sc_gather():
  return functools.partial(
      pl.kernel,
      out_type=jax.ShapeDtypeStruct((B * _ROWPAD,), jnp.float32),
      mesh=plsc.VectorSubcoreMesh(core_axis_name="c", subcore_axis_name="s",
                                  num_cores=_NC),
      scratch_types=[
          pltpu.VMEM((_NIDX,), jnp.int32),
          pltpu.VMEM((_TAB_LEN,), jnp.float32),
          pltpu.VMEM((8 * _CAT,), jnp.int32),
          pltpu.VMEM((8 * _CAT,), jnp.int32),
          pltpu.VMEM((2 * _CAT,), jnp.int32),
          pltpu.VMEM((_NOUT,), jnp.float32),
      ],
      compiler_params=pltpu.CompilerParams(use_tc_tiling_on_sc=False,
                                           needs_layout_passes=False),
  )(_sc_gather_body)


_BBLK = 1024


def _tc_body(cat_ref, pnum_ref, pdesc_ref, wdesc_ref, bdesc_ref,
             w1a_ref, w1b_ref, w1c_ref, b1_ref, w2_ref, b2_ref,
             w3_ref, b3_ref, out_ref):
  pd = pdesc_ref[...]
  ss = jnp.sum(pd * pd, axis=1, keepdims=True)
  inv = 1.0 / (jnp.sqrt(ss) + 1e-08)
  d0 = jnp.dot(pd, wdesc_ref[...], preferred_element_type=jnp.float32)
  desc = d0 * inv + bdesc_ref[...]
  cat = cat_ref[...].reshape(_BBLK, _ROWPAD)[:, :_CAT]
  h = (jnp.dot(cat, w1a_ref[...], preferred_element_type=jnp.float32)
       + jnp.dot(pnum_ref[...], w1b_ref[...], preferred_element_type=jnp.float32)
       + jnp.dot(desc, w1c_ref[...], preferred_element_type=jnp.float32)
       + b1_ref[...])
  h = jnp.maximum(h, 0.0)
  h = jnp.maximum(
      jnp.dot(h, w2_ref[...], preferred_element_type=jnp.float32) + b2_ref[...],
      0.0)
  out_ref[...] = (
      jnp.dot(h, w3_ref[...], preferred_element_type=jnp.float32) + b3_ref[...])


def _full(shape):
  return pl.BlockSpec(shape, lambda i: (0,) * len(shape))


def _tc_mlp(cat3, p_num, p_desc, W_desc, b_desc, W1a, W1b, W1c, b1, W2, b2,
            W3, b3):
  return pl.pallas_call(
      _tc_body,
      grid=(B // _BBLK,),
      in_specs=[
          pl.BlockSpec((_BBLK // 8, 8, _ROWPAD), lambda i: (i, 0, 0)),
          pl.BlockSpec((_BBLK, 2), lambda i: (i, 0)),
          pl.BlockSpec((_BBLK, DESC_IN), lambda i: (i, 0)),
          _full((DESC_IN, DESC_OUT)),
          _full((1, DESC_OUT)),
          _full((_CAT, H1)),
          _full((2, H1)),
          _full((DESC_OUT, H1)),
          _full((1, H1)),
          _full((H1, H2)),
          _full((1, H2)),
          _full((H2, OUT)),
          _full((1, OUT)),
      ],
      out_specs=pl.BlockSpec((_BBLK, OUT), lambda i: (i, 0)),
      out_shape=jax.ShapeDtypeStruct((B, OUT), jnp.float32),
  )(cat3, p_num, p_desc, W_desc, b_desc, W1a, W1b, W1c, b1, W2, b2, W3, b3)


def kernel(p_cat, p_num, p_desc, t_light, t_tol, t_hum, t_water, t_care,
           t_size, t_climate, W_desc, b_desc, W1, b1, W2, b2, W3, b3):
  pcat_w = p_cat.astype(jnp.int32).reshape(_NW, _NIDX)
  tab_flat = jnp.concatenate(
      [t.reshape(-1) for t in
       (t_light, t_tol, t_hum, t_water, t_care, t_size, t_climate)])
  cat_flat = _sc_gather()(pcat_w, tab_flat, jnp.asarray(_PIDX_H),
                          jnp.asarray(_DST_H), jnp.asarray(_FO_H))
  cat3 = cat_flat.reshape(B // 8, 8, _ROWPAD)
  W1a = W1[:_CAT]
  W1b = W1[_CAT:_CAT + 2]
  W1c = W1[_CAT + 2:]
  return _tc_mlp(cat3, p_num, p_desc, W_desc, b_desc.reshape(1, -1), W1a,
                 W1b, W1c, b1.reshape(1, -1), W2, b2.reshape(1, -1), W3,
                 b3.reshape(1, -1))
